# Initial kernel scaffold; baseline (speedup 1.0000x reference)
import jax
import jax.numpy as jnp
from jax.experimental import pallas as pl


def kernel(x, edge_index, W1l, b1, W1r, W2l, b2, W2r):
    raise NotImplementedError("write your pallas kernel here")


# speedup vs baseline: 5.0044x; 5.0044x over previous
import functools

import jax
import jax.numpy as jnp
from jax import lax
from jax.experimental import pallas as pl
from jax.experimental.pallas import tpu as pltpu
from jax.experimental.pallas import tpu_sc as plsc

NC = 2
NS = 16
K = 80


def _sc_agg(x, src, dst, z2, z1):
    N, D = x.shape
    NP = z2.shape[0]
    E = src.shape[0]
    NW = NC * NS
    ept = E // NW
    rpt = NP // NS
    assert E % NW == 0 and ept % K == 0 and NP % NS == 0 and rpt % 8 == 0

    mesh = plsc.VectorSubcoreMesh(core_axis_name="c", subcore_axis_name="s")

    @functools.partial(
        pl.kernel,
        mesh=mesh,
        out_type=[
            jax.ShapeDtypeStruct((NC, NP, D), jnp.float32),
            jax.ShapeDtypeStruct((NP,), jnp.float32),
            jax.ShapeDtypeStruct((NP,), jnp.float32),
        ],
        scratch_types=[
            pltpu.VMEM((K,), jnp.int32),
            pltpu.VMEM((K,), jnp.int32),
            pltpu.VMEM((K, D), jnp.float32),
            pltpu.VMEM((K,), jnp.float32),
            pltpu.VMEM_SHARED((NP, D), jnp.float32),
            pltpu.VMEM_SHARED((NP,), jnp.float32),
            pltpu.SemaphoreType.DMA,
        ],
    )
    def agg(x_hbm, src_hbm, dst_hbm, z2_hbm, z1_hbm, out_hbm, c0_hbm, c1_hbm,
            sidx, didx, rows, ones, acc, cacc, sem):
        c = lax.axis_index("c")
        s = lax.axis_index("s")
        wid = c * NS + s

        pltpu.sync_copy(z2_hbm.at[pl.ds(s * rpt, rpt)],
                        acc.at[pl.ds(s * rpt, rpt)])

        @pl.when(s == 0)
        def _():
            pltpu.sync_copy(z1_hbm, cacc)

        for j in range(K // 16):
            ones[pl.ds(j * 16, 16)] = jnp.ones((16,), jnp.float32)

        plsc.subcore_barrier()

        base0 = wid * ept

        def chunk(i, carry):
            b = base0 + i * K
            pltpu.sync_copy(src_hbm.at[pl.ds(b, K)], sidx)
            pltpu.sync_copy(dst_hbm.at[pl.ds(b, K)], didx)
            pltpu.async_copy(x_hbm.at[sidx], rows, sem).wait()
            pltpu.sync_copy(rows, acc.at[didx], add=True)
            pltpu.sync_copy(ones, cacc.at[didx], add=True)
            return carry

        lax.fori_loop(0, ept // K, chunk, 0)

        plsc.subcore_barrier()

        pltpu.sync_copy(acc.at[pl.ds(s * rpt, rpt)],
                        out_hbm.at[c, pl.ds(s * rpt, rpt)])

        @pl.when(jnp.logical_and(s == 0, c == 0))
        def _():
            pltpu.sync_copy(cacc, c0_hbm)

        @pl.when(jnp.logical_and(s == 0, c == 1))
        def _():
            pltpu.sync_copy(cacc, c1_hbm)

    return agg(x, src, dst, z2, z1)


def _tc_layer(parts, cnts, x, wlt, b2d, wrt, relu):
    NP, D = x.shape
    BN = 2048
    assert NP % BN == 0
    grid = NP // BN

    def body(part_ref, cnt_ref, x_ref, wl_ref, b_ref, wr_ref, o_ref):
        i = pl.program_id(0)
        csum = (cnt_ref[0, pl.ds(i * BN, BN)] +
                cnt_ref[1, pl.ds(i * BN, BN)])
        inv = 1.0 / jnp.maximum(csum, 1.0)
        agg = part_ref[0] + part_ref[1]
        mean = agg * inv[:, None]
        h = (jnp.dot(mean, wl_ref[...], preferred_element_type=jnp.float32)
             + b_ref[...]
             + jnp.dot(x_ref[...], wr_ref[...],
                       preferred_element_type=jnp.float32))
        if relu:
            h = jnp.maximum(h, 0.0)
        o_ref[...] = h

    return pl.pallas_call(
        body,
        grid=(grid,),
        in_specs=[
            pl.BlockSpec((2, BN, D), lambda i: (0, i, 0)),
            pl.BlockSpec((2, NP), lambda i: (0, 0)),
            pl.BlockSpec((BN, D), lambda i: (i, 0)),
            pl.BlockSpec((D, D), lambda i: (0, 0)),
            pl.BlockSpec((1, D), lambda i: (0, 0)),
            pl.BlockSpec((D, D), lambda i: (0, 0)),
        ],
        out_specs=pl.BlockSpec((BN, D), lambda i: (i, 0)),
        out_shape=jax.ShapeDtypeStruct((NP, D), jnp.float32),
    )(parts, cnts, x, wlt, b2d, wrt)


@jax.jit
def kernel(x, edge_index, W1l, b1, W1r, W2l, b2, W2r):
    N, D = x.shape
    NP = ((N + 2047) // 2048) * 2048
    src = edge_index[0]
    dst = edge_index[1]
    xp = jnp.pad(x, ((0, NP - N), (0, 0)))
    z2 = jnp.zeros((NP, D), jnp.float32)
    z1 = jnp.zeros((NP,), jnp.float32)

    p1, c1a, c1b = _sc_agg(xp, src, dst, z2, z1)
    cn1 = jnp.stack([c1a, c1b])
    h = _tc_layer(p1, cn1, xp, W1l.T, b1.reshape(1, D), W1r.T, relu=True)
    p2, c2a, c2b = _sc_agg(h, src, dst, z2, z1)
    cn2 = jnp.stack([c2a, c2b])
    out = _tc_layer(p2, cn2, h, W2l.T, b2.reshape(1, D), W2r.T, relu=False)
    return out[:N]


# speedup vs baseline: 11.0427x; 2.2066x over previous
import functools

import jax
import jax.numpy as jnp
from jax import lax
from jax.experimental import pallas as pl
from jax.experimental.pallas import tpu as pltpu
from jax.experimental.pallas import tpu_sc as plsc

NC = 2
NS = 16
K = 80


def _sc_agg(x, src, dst, z2):
    NP, D = x.shape
    NW = NC * NS
    E = src.shape[0]
    ept = E // NW
    n_chunks = ept // K
    rpt = NP // NS
    assert E % NW == 0 and ept % K == 0 and rpt % 8 == 0
    assert n_chunks % 2 == 1 and n_chunks >= 3

    mesh = plsc.VectorSubcoreMesh(core_axis_name="c", subcore_axis_name="s")

    @functools.partial(
        pl.kernel, mesh=mesh,
        out_type=jax.ShapeDtypeStruct((NC, NP, D), jnp.float32),
        scratch_types=[
            pltpu.VMEM((ept,), jnp.int32),
            pltpu.VMEM((ept,), jnp.int32),
            pltpu.VMEM((1, K), jnp.int32),
            pltpu.VMEM((K, D), jnp.float32),
            pltpu.VMEM((K, D), jnp.float32),
            pltpu.VMEM_SHARED((NP, D), jnp.float32),
            pltpu.SemaphoreType.DMA,
            pltpu.SemaphoreType.DMA,
        ],
    )
    def agg(x_hbm, src_hbm, dst_hbm, z2_hbm, out_hbm,
            sidx, didx, didxs, rowsa, rowsb, acc, sema, semb):
        c = lax.axis_index("c")
        s = lax.axis_index("s")
        wid = c * NS + s

        pltpu.sync_copy(z2_hbm.at[pl.ds(s * rpt, rpt)],
                        acc.at[pl.ds(s * rpt, rpt)])

        pltpu.sync_copy(src_hbm.at[pl.ds(wid * ept, ept)], sidx)
        pltpu.sync_copy(dst_hbm.at[pl.ds(wid * ept, ept)], didx)

        plsc.subcore_barrier()

        def scatter(ci, rows):
            for j in range(K // 16):
                didxs[0, pl.ds(j * 16, 16)] = didx[pl.ds(ci * K + j * 16, 16)]
            pltpu.sync_copy(rows, acc.at[didxs.at[0]], add=True)

        def fire(ci, rows, sem):
            pltpu.async_copy(x_hbm.at[sidx.at[pl.ds(ci * K, K)]], rows, sem)

        def drain(rows, sem):
            pltpu.make_async_copy(x_hbm.at[sidx.at[pl.ds(0, K)]],
                                  rows, sem).wait()

        fire(0, rowsa, sema)

        def pair(g, carry):
            c0 = 2 * g
            drain(rowsa, sema)
            fire(c0 + 1, rowsb, semb)
            scatter(c0, rowsa)
            fire(c0 + 2, rowsa, sema)
            drain(rowsb, semb)
            scatter(c0 + 1, rowsb)
            return carry

        lax.fori_loop(0, (n_chunks - 1) // 2, pair, 0)

        drain(rowsa, sema)
        scatter(n_chunks - 1, rowsa)

        plsc.subcore_barrier()

        pltpu.sync_copy(acc.at[pl.ds(s * rpt, rpt)],
                        out_hbm.at[c, pl.ds(s * rpt, rpt)])

    return agg(x, src, dst, z2)


def _sc_counts(dstr, z1):
    NP = z1.shape[0]
    NW = NC * NS
    n_chunks = dstr.shape[1]
    assert dstr.shape == (NW, n_chunks, K)

    mesh = plsc.VectorSubcoreMesh(core_axis_name="c", subcore_axis_name="s")

    @functools.partial(
        pl.kernel, mesh=mesh,
        out_type=[jax.ShapeDtypeStruct((NP,), jnp.float32),
                  jax.ShapeDtypeStruct((NP,), jnp.float32)],
        scratch_types=[
            pltpu.VMEM((n_chunks, K), jnp.int32),
            pltpu.VMEM((K,), jnp.float32),
            pltpu.VMEM_SHARED((NP,), jnp.float32),
            pltpu.SemaphoreType.DMA,
        ],
    )
    def cnt(dst_hbm, z1_hbm, c0_hbm, c1_hbm, didx, ones, cacc, sem):
        c = lax.axis_index("c")
        s = lax.axis_index("s")
        wid = c * NS + s

        @pl.when(s == 0)
        def _():
            pltpu.sync_copy(z1_hbm, cacc)

        for j in range(K // 16):
            ones[pl.ds(j * 16, 16)] = jnp.ones((16,), jnp.float32)

        pltpu.sync_copy(dst_hbm.at[wid], didx)

        plsc.subcore_barrier()

        def fire(i, carry):
            pltpu.sync_copy(ones, cacc.at[didx.at[i]], add=True)
            return carry

        lax.fori_loop(0, n_chunks, fire, 0)

        plsc.subcore_barrier()

        @pl.when(jnp.logical_and(s == 0, c == 0))
        def _():
            pltpu.sync_copy(cacc, c0_hbm)

        @pl.when(jnp.logical_and(s == 0, c == 1))
        def _():
            pltpu.sync_copy(cacc, c1_hbm)

    return cnt(dstr, z1)


def _tc_layer(parts, cnts, x, wlt, b2d, wrt, relu):
    NP, D = x.shape
    BN = 2048
    assert NP % BN == 0
    grid = NP // BN

    def body(part_ref, cnt_ref, x_ref, wl_ref, b_ref, wr_ref, o_ref):
        i = pl.program_id(0)
        csum = (cnt_ref[0, pl.ds(i * BN, BN)] +
                cnt_ref[1, pl.ds(i * BN, BN)])
        inv = 1.0 / jnp.maximum(csum, 1.0)
        agg = part_ref[0] + part_ref[1]
        mean = agg * inv[:, None]
        h = (jnp.dot(mean, wl_ref[...], preferred_element_type=jnp.float32)
             + b_ref[...]
             + jnp.dot(x_ref[...], wr_ref[...],
                       preferred_element_type=jnp.float32))
        if relu:
            h = jnp.maximum(h, 0.0)
        o_ref[...] = h

    return pl.pallas_call(
        body,
        grid=(grid,),
        in_specs=[
            pl.BlockSpec((2, BN, D), lambda i: (0, i, 0)),
            pl.BlockSpec((2, NP), lambda i: (0, 0)),
            pl.BlockSpec((BN, D), lambda i: (i, 0)),
            pl.BlockSpec((D, D), lambda i: (0, 0)),
            pl.BlockSpec((1, D), lambda i: (0, 0)),
            pl.BlockSpec((D, D), lambda i: (0, 0)),
        ],
        out_specs=pl.BlockSpec((BN, D), lambda i: (i, 0)),
        out_shape=jax.ShapeDtypeStruct((NP, D), jnp.float32),
    )(parts, cnts, x, wlt, b2d, wrt)


@jax.jit
def kernel(x, edge_index, W1l, b1, W1r, W2l, b2, W2r):
    N, D = x.shape
    NP = ((N + 2047) // 2048) * 2048
    NW = NC * NS
    E = edge_index.shape[1]
    n_chunks = E // (NW * K)
    src = edge_index[0]
    dst = edge_index[1]
    dstr = dst.reshape(NW, n_chunks, K)
    xp = jnp.pad(x, ((0, NP - N), (0, 0)))
    z2 = jnp.zeros((NP, D), jnp.float32)
    z1 = jnp.zeros((NP,), jnp.float32)

    c1a, c1b = _sc_counts(dstr, z1)
    cn1 = jnp.stack([c1a, c1b])
    p1 = _sc_agg(xp, src, dst, z2)
    h = _tc_layer(p1, cn1, xp, W1l.T, b1.reshape(1, D), W1r.T, relu=True)
    p2 = _sc_agg(h, src, dst, z2)
    out = _tc_layer(p2, cn1, h, W2l.T, b2.reshape(1, D), W2r.T, relu=False)
    return out[:N]


# speedup vs baseline: 11.5203x; 1.0433x over previous
import functools

import jax
import jax.numpy as jnp
from jax import lax
from jax.experimental import pallas as pl
from jax.experimental.pallas import tpu as pltpu
from jax.experimental.pallas import tpu_sc as plsc

NC = 2
NS = 16
K = 80


def _sc_agg(x, src, dst, z2, z1, with_counts):
    NP, D = x.shape
    NW = NC * NS
    E = src.shape[0]
    ept = E // NW
    n_chunks = ept // K
    rpt = NP // NS
    assert E % NW == 0 and ept % K == 0 and rpt % 8 == 0
    assert n_chunks % 2 == 1 and n_chunks >= 3

    mesh = plsc.VectorSubcoreMesh(core_axis_name="c", subcore_axis_name="s")

    out_type = [jax.ShapeDtypeStruct((NC, NP, D), jnp.float32)]
    if with_counts:
        out_type += [jax.ShapeDtypeStruct((NP,), jnp.float32),
                     jax.ShapeDtypeStruct((NP,), jnp.float32)]

    @functools.partial(
        pl.kernel, mesh=mesh, out_type=out_type,
        scratch_types=[
            pltpu.VMEM((ept,), jnp.int32),
            pltpu.VMEM((ept,), jnp.int32),
            pltpu.VMEM((1, K), jnp.int32),
            pltpu.VMEM((1, K), jnp.int32),
            pltpu.VMEM((K, D), jnp.float32),
            pltpu.VMEM((K, D), jnp.float32),
            pltpu.VMEM((K,), jnp.float32),
            pltpu.VMEM_SHARED((NP, D), jnp.float32),
            pltpu.VMEM_SHARED((NP,), jnp.float32),
            pltpu.SemaphoreType.DMA,
            pltpu.SemaphoreType.DMA,
            pltpu.SemaphoreType.DMA,
            pltpu.SemaphoreType.DMA,
        ],
    )
    def agg(*refs):
        if with_counts:
            (x_hbm, src_hbm, dst_hbm, z2_hbm, z1_hbm,
             out_hbm, c0_hbm, c1_hbm,
             sidx, didx, didxsa, didxsb, rowsa, rowsb, ones,
             acc, cacc, sema, semb, csema, csemb) = refs
        else:
            (x_hbm, src_hbm, dst_hbm, z2_hbm, z1_hbm, out_hbm,
             sidx, didx, didxsa, didxsb, rowsa, rowsb, ones,
             acc, cacc, sema, semb, csema, csemb) = refs
        c = lax.axis_index("c")
        s = lax.axis_index("s")
        wid = c * NS + s

        pltpu.sync_copy(z2_hbm.at[pl.ds(s * rpt, rpt)],
                        acc.at[pl.ds(s * rpt, rpt)])
        if with_counts:
            @pl.when(s == 0)
            def _():
                pltpu.sync_copy(z1_hbm, cacc)

            for j in range(K // 16):
                ones[pl.ds(j * 16, 16)] = jnp.ones((16,), jnp.float32)

        pltpu.sync_copy(src_hbm.at[pl.ds(wid * ept, ept)], sidx)
        pltpu.sync_copy(dst_hbm.at[pl.ds(wid * ept, ept)], didx)

        plsc.subcore_barrier()

        def scatter(ci, rows, didxs):
            for j in range(K // 16):
                didxs[0, pl.ds(j * 16, 16)] = didx[pl.ds(ci * K + j * 16, 16)]
            pltpu.sync_copy(rows, acc.at[didxs.at[0]], add=True)

        def cfire(didxs, csem):
            pltpu.async_copy(ones, cacc.at[didxs.at[0]], csem, add=True)

        def cdrain(didxs, csem):
            pltpu.make_async_copy(ones, cacc.at[didxs.at[0]], csem).wait()

        def fire(ci, rows, sem):
            pltpu.async_copy(x_hbm.at[sidx.at[pl.ds(ci * K, K)]], rows, sem)

        def drain(rows, sem):
            pltpu.make_async_copy(x_hbm.at[sidx.at[pl.ds(0, K)]],
                                  rows, sem).wait()

        fire(0, rowsa, sema)

        def pair(g, carry):
            c0 = 2 * g
            drain(rowsa, sema)
            fire(c0 + 1, rowsb, semb)
            if with_counts:
                @pl.when(g > 0)
                def _():
                    cdrain(didxsa, csema)
            scatter(c0, rowsa, didxsa)
            if with_counts:
                cfire(didxsa, csema)
            fire(c0 + 2, rowsa, sema)
            drain(rowsb, semb)
            if with_counts:
                @pl.when(g > 0)
                def _():
                    cdrain(didxsb, csemb)
            scatter(c0 + 1, rowsb, didxsb)
            if with_counts:
                cfire(didxsb, csemb)
            return carry

        lax.fori_loop(0, (n_chunks - 1) // 2, pair, 0)

        drain(rowsa, sema)
        if with_counts:
            cdrain(didxsa, csema)
        scatter(n_chunks - 1, rowsa, didxsa)
        if with_counts:
            cfire(didxsa, csema)
            cdrain(didxsa, csema)
            cdrain(didxsb, csemb)

        plsc.subcore_barrier()

        pltpu.sync_copy(acc.at[pl.ds(s * rpt, rpt)],
                        out_hbm.at[c, pl.ds(s * rpt, rpt)])

        if with_counts:
            @pl.when(jnp.logical_and(s == 0, c == 0))
            def _():
                pltpu.sync_copy(cacc, c0_hbm)

            @pl.when(jnp.logical_and(s == 0, c == 1))
            def _():
                pltpu.sync_copy(cacc, c1_hbm)

    res = agg(x, src, dst, z2, z1)
    if not with_counts and isinstance(res, (list, tuple)):
        return res[0]
    return res


def _tc_layer(parts, cnts, x, wlt, b2d, wrt, relu):
    NP, D = x.shape
    BN = 2048
    assert NP % BN == 0
    grid = NP // BN

    def body(part_ref, cnt_ref, x_ref, wl_ref, b_ref, wr_ref, o_ref):
        i = pl.program_id(0)
        csum = (cnt_ref[0, pl.ds(i * BN, BN)] +
                cnt_ref[1, pl.ds(i * BN, BN)])
        inv = 1.0 / jnp.maximum(csum, 1.0)
        agg = part_ref[0] + part_ref[1]
        mean = agg * inv[:, None]
        h = (jnp.dot(mean, wl_ref[...], preferred_element_type=jnp.float32)
             + b_ref[...]
             + jnp.dot(x_ref[...], wr_ref[...],
                       preferred_element_type=jnp.float32))
        if relu:
            h = jnp.maximum(h, 0.0)
        o_ref[...] = h

    return pl.pallas_call(
        body,
        grid=(grid,),
        in_specs=[
            pl.BlockSpec((2, BN, D), lambda i: (0, i, 0)),
            pl.BlockSpec((2, NP), lambda i: (0, 0)),
            pl.BlockSpec((BN, D), lambda i: (i, 0)),
            pl.BlockSpec((D, D), lambda i: (0, 0)),
            pl.BlockSpec((1, D), lambda i: (0, 0)),
            pl.BlockSpec((D, D), lambda i: (0, 0)),
        ],
        out_specs=pl.BlockSpec((BN, D), lambda i: (i, 0)),
        out_shape=jax.ShapeDtypeStruct((NP, D), jnp.float32),
    )(parts, cnts, x, wlt, b2d, wrt)


@jax.jit
def kernel(x, edge_index, W1l, b1, W1r, W2l, b2, W2r):
    N, D = x.shape
    NP = ((N + 2047) // 2048) * 2048
    NW = NC * NS
    E = edge_index.shape[1]
    n_chunks = E // (NW * K)
    src = edge_index[0]
    dst = edge_index[1]
    xp = jnp.pad(x, ((0, NP - N), (0, 0)))
    z2 = jnp.zeros((NP, D), jnp.float32)
    z1 = jnp.zeros((NP,), jnp.float32)

    p1, c1a, c1b = _sc_agg(xp, src, dst, z2, z1, with_counts=True)
    cn1 = jnp.stack([c1a, c1b])
    h = _tc_layer(p1, cn1, xp, W1l.T, b1.reshape(1, D), W1r.T, relu=True)
    p2 = _sc_agg(h, src, dst, z2, z1, with_counts=False)
    out = _tc_layer(p2, cn1, h, W2l.T, b2.reshape(1, D), W2r.T, relu=False)
    return out[:N]


# speedup vs baseline: 11.6760x; 1.0135x over previous
import functools

import jax
import jax.numpy as jnp
from jax import lax
from jax.experimental import pallas as pl
from jax.experimental.pallas import tpu as pltpu
from jax.experimental.pallas import tpu_sc as plsc

NC = 2
NS = 16
K = 80


def _sc_agg(x, src, dst, z2, z1, with_counts):
    NP, D = x.shape
    NW = NC * NS
    E = src.shape[0]
    ept = E // NW
    n_chunks = ept // K
    rpt = NP // NS
    assert E % NW == 0 and ept % K == 0 and rpt % 8 == 0
    assert n_chunks % 2 == 1 and n_chunks >= 3

    mesh = plsc.VectorSubcoreMesh(core_axis_name="c", subcore_axis_name="s")

    out_type = [jax.ShapeDtypeStruct((NC, NP, D), jnp.float32)]
    if with_counts:
        out_type += [jax.ShapeDtypeStruct((NP,), jnp.float32),
                     jax.ShapeDtypeStruct((NP,), jnp.float32)]

    @functools.partial(
        pl.kernel, mesh=mesh, out_type=out_type,
        scratch_types=[
            pltpu.VMEM((ept,), jnp.int32),
            pltpu.VMEM((ept,), jnp.int32),
            pltpu.VMEM((1, K), jnp.int32),
            pltpu.VMEM((1, K), jnp.int32),
            pltpu.VMEM((K, D), jnp.float32),
            pltpu.VMEM((K, D), jnp.float32),
            pltpu.VMEM((K,), jnp.float32),
            pltpu.VMEM_SHARED((NP, D), jnp.float32),
            pltpu.VMEM_SHARED((NP,), jnp.float32),
            pltpu.SemaphoreType.DMA,
            pltpu.SemaphoreType.DMA,
            pltpu.SemaphoreType.DMA,
            pltpu.SemaphoreType.DMA,
        ],
    )
    def agg(*refs):
        if with_counts:
            (x_hbm, src_hbm, dst_hbm, z2_hbm, z1_hbm,
             out_hbm, c0_hbm, c1_hbm,
             sidx, didx, didxsa, didxsb, rowsa, rowsb, ones,
             acc, cacc, sema, semb, csema, csemb) = refs
        else:
            (x_hbm, src_hbm, dst_hbm, z2_hbm, z1_hbm, out_hbm,
             sidx, didx, didxsa, didxsb, rowsa, rowsb, ones,
             acc, cacc, sema, semb, csema, csemb) = refs
        c = lax.axis_index("c")
        s = lax.axis_index("s")
        wid = c * NS + s

        zcp = pltpu.async_copy(z2_hbm.at[pl.ds(s * rpt, rpt)],
                               acc.at[pl.ds(s * rpt, rpt)], sema)
        scp = pltpu.async_copy(src_hbm.at[pl.ds(wid * ept, ept)], sidx, semb)
        dcp = pltpu.async_copy(dst_hbm.at[pl.ds(wid * ept, ept)], didx, csema)
        if with_counts:
            @pl.when(s == 0)
            def _():
                pltpu.sync_copy(z1_hbm, cacc)

            for j in range(K // 16):
                ones[pl.ds(j * 16, 16)] = jnp.ones((16,), jnp.float32)
        zcp.wait()
        scp.wait()
        dcp.wait()

        plsc.subcore_barrier()

        def scatter(ci, rows, didxs):
            for j in range(K // 16):
                didxs[0, pl.ds(j * 16, 16)] = didx[pl.ds(ci * K + j * 16, 16)]
            pltpu.sync_copy(rows, acc.at[didxs.at[0]], add=True)

        def cfire(didxs, csem):
            pltpu.async_copy(ones, cacc.at[didxs.at[0]], csem, add=True)

        def cdrain(didxs, csem):
            pltpu.make_async_copy(ones, cacc.at[didxs.at[0]], csem).wait()

        def fire(ci, rows, sem):
            pltpu.async_copy(x_hbm.at[sidx.at[pl.ds(ci * K, K)]], rows, sem)

        def drain(rows, sem):
            pltpu.make_async_copy(x_hbm.at[sidx.at[pl.ds(0, K)]],
                                  rows, sem).wait()

        fire(0, rowsa, sema)

        def pair(g, carry):
            c0 = 2 * g
            drain(rowsa, sema)
            fire(c0 + 1, rowsb, semb)
            if with_counts:
                @pl.when(g > 0)
                def _():
                    cdrain(didxsa, csema)
            scatter(c0, rowsa, didxsa)
            if with_counts:
                cfire(didxsa, csema)
            fire(c0 + 2, rowsa, sema)
            drain(rowsb, semb)
            if with_counts:
                @pl.when(g > 0)
                def _():
                    cdrain(didxsb, csemb)
            scatter(c0 + 1, rowsb, didxsb)
            if with_counts:
                cfire(didxsb, csemb)
            return carry

        lax.fori_loop(0, (n_chunks - 1) // 2, pair, 0)

        drain(rowsa, sema)
        if with_counts:
            cdrain(didxsa, csema)
        scatter(n_chunks - 1, rowsa, didxsa)
        if with_counts:
            cfire(didxsa, csema)
            cdrain(didxsa, csema)
            cdrain(didxsb, csemb)

        plsc.subcore_barrier()

        pltpu.sync_copy(acc.at[pl.ds(s * rpt, rpt)],
                        out_hbm.at[c, pl.ds(s * rpt, rpt)])

        if with_counts:
            @pl.when(jnp.logical_and(s == 0, c == 0))
            def _():
                pltpu.sync_copy(cacc, c0_hbm)

            @pl.when(jnp.logical_and(s == 0, c == 1))
            def _():
                pltpu.sync_copy(cacc, c1_hbm)

    res = agg(x, src, dst, z2, z1)
    if not with_counts and isinstance(res, (list, tuple)):
        return res[0]
    return res


def _tc_layer(parts, cnts, x, wlt, b2d, wrt, relu):
    NP, D = x.shape
    BN = 2048
    assert NP % BN == 0
    grid = NP // BN

    def body(part_ref, cnt_ref, x_ref, wl_ref, b_ref, wr_ref, o_ref):
        i = pl.program_id(0)
        csum = (cnt_ref[0, pl.ds(i * BN, BN)] +
                cnt_ref[1, pl.ds(i * BN, BN)])
        inv = 1.0 / jnp.maximum(csum, 1.0)
        agg = part_ref[0] + part_ref[1]
        mean = agg * inv[:, None]
        h = (jnp.dot(mean, wl_ref[...], preferred_element_type=jnp.float32)
             + b_ref[...]
             + jnp.dot(x_ref[...], wr_ref[...],
                       preferred_element_type=jnp.float32))
        if relu:
            h = jnp.maximum(h, 0.0)
        o_ref[...] = h

    return pl.pallas_call(
        body,
        grid=(grid,),
        in_specs=[
            pl.BlockSpec((2, BN, D), lambda i: (0, i, 0)),
            pl.BlockSpec((2, NP), lambda i: (0, 0)),
            pl.BlockSpec((BN, D), lambda i: (i, 0)),
            pl.BlockSpec((D, D), lambda i: (0, 0)),
            pl.BlockSpec((1, D), lambda i: (0, 0)),
            pl.BlockSpec((D, D), lambda i: (0, 0)),
        ],
        out_specs=pl.BlockSpec((BN, D), lambda i: (i, 0)),
        out_shape=jax.ShapeDtypeStruct((NP, D), jnp.float32),
    )(parts, cnts, x, wlt, b2d, wrt)


@jax.jit
def kernel(x, edge_index, W1l, b1, W1r, W2l, b2, W2r):
    N, D = x.shape
    NP = ((N + 2047) // 2048) * 2048
    NW = NC * NS
    E = edge_index.shape[1]
    n_chunks = E // (NW * K)
    src = edge_index[0]
    dst = edge_index[1]
    xp = jnp.pad(x, ((0, NP - N), (0, 0)))
    z2 = jnp.zeros((NP, D), jnp.float32)
    z1 = jnp.zeros((NP,), jnp.float32)

    p1, c1a, c1b = _sc_agg(xp, src, dst, z2, z1, with_counts=True)
    cn1 = jnp.stack([c1a, c1b])
    h = _tc_layer(p1, cn1, xp, W1l.T, b1.reshape(1, D), W1r.T, relu=True)
    p2 = _sc_agg(h, src, dst, z2, z1, with_counts=False)
    out = _tc_layer(p2, cn1, h, W2l.T, b2.reshape(1, D), W2r.T, relu=False)
    return out[:N]


# speedup vs baseline: 13.3338x; 1.1420x over previous
import functools

import jax
import jax.numpy as jnp
from jax import lax
from jax.experimental import pallas as pl
from jax.experimental.pallas import tpu as pltpu
from jax.experimental.pallas import tpu_sc as plsc

NC = 2
NS = 16
K = 80


def _sc_agg(x, src, dstr3, z2, z1, with_counts):
    NP, D = x.shape
    NW = NC * NS
    E = src.shape[0]
    ept = E // NW
    n_chunks = ept // K
    rpt = NP // NS
    assert E % NW == 0 and ept % K == 0 and rpt % 8 == 0
    assert dstr3.shape == (NW * n_chunks, 1, K)
    assert n_chunks % 3 == 2 and n_chunks >= 8

    mesh = plsc.VectorSubcoreMesh(core_axis_name="c", subcore_axis_name="s")

    out_type = [jax.ShapeDtypeStruct((NC, NP, D), jnp.float32)]
    if with_counts:
        out_type += [jax.ShapeDtypeStruct((NP,), jnp.float32),
                     jax.ShapeDtypeStruct((NP,), jnp.float32)]

    @functools.partial(
        pl.kernel, mesh=mesh, out_type=out_type,
        scratch_types=[
            pltpu.VMEM((ept,), jnp.int32),
            pltpu.VMEM((1, K), jnp.int32),
            pltpu.VMEM((1, K), jnp.int32),
            pltpu.VMEM((1, K), jnp.int32),
            pltpu.VMEM((K, D), jnp.float32),
            pltpu.VMEM((K, D), jnp.float32),
            pltpu.VMEM((K, D), jnp.float32),
            pltpu.VMEM((K,), jnp.float32),
            pltpu.VMEM_SHARED((NP, D), jnp.float32),
            pltpu.VMEM_SHARED((NP,), jnp.float32),
        ] + [pltpu.SemaphoreType.DMA] * 12,
    )
    def agg(*refs):
        if with_counts:
            (x_hbm, src_hbm, dst_hbm, z2_hbm, z1_hbm,
             out_hbm, c0_hbm, c1_hbm, sidx,
             dx0, dx1, dx2, rw0, rw1, rw2, ones, acc, cacc, *sems) = refs
        else:
            (x_hbm, src_hbm, dst_hbm, z2_hbm, z1_hbm, out_hbm, sidx,
             dx0, dx1, dx2, rw0, rw1, rw2, ones, acc, cacc, *sems) = refs
        didxs = [dx0, dx1, dx2]
        rows = [rw0, rw1, rw2]
        gsem = sems[0:3]
        ssem = sems[3:6]
        isem = sems[6:9]
        csem = sems[9:12]
        c = lax.axis_index("c")
        s = lax.axis_index("s")
        wid = c * NS + s
        gbase = wid * n_chunks

        zcp = pltpu.async_copy(z2_hbm.at[pl.ds(s * rpt, rpt)],
                               acc.at[pl.ds(s * rpt, rpt)], gsem[0])
        scp = pltpu.async_copy(src_hbm.at[pl.ds(wid * ept, ept)],
                               sidx, gsem[1])
        if with_counts:
            @pl.when(s == 0)
            def _():
                pltpu.sync_copy(z1_hbm, cacc)

            for j in range(K // 16):
                ones[pl.ds(j * 16, 16)] = jnp.ones((16,), jnp.float32)
        zcp.wait()
        scp.wait()

        plsc.subcore_barrier()

        def ifire(ci, b):
            pltpu.async_copy(dst_hbm.at[gbase + ci], didxs[b], isem[b])

        def idrain(b):
            pltpu.make_async_copy(dst_hbm.at[0], didxs[b], isem[b]).wait()

        def gfire(ci, b):
            pltpu.async_copy(x_hbm.at[sidx.at[pl.ds(ci * K, K)]],
                             rows[b], gsem[b])

        def gdrain(b):
            pltpu.make_async_copy(x_hbm.at[sidx.at[pl.ds(0, K)]],
                                  rows[b], gsem[b]).wait()

        def sfire(b):
            pltpu.async_copy(rows[b], acc.at[didxs[b].at[0]], ssem[b],
                             add=True)

        def sdrain(b):
            pltpu.make_async_copy(rows[b], acc.at[didxs[b].at[0]],
                                  ssem[b]).wait()

        def cfire(b):
            pltpu.async_copy(ones, cacc.at[didxs[b].at[0]], csem[b],
                             add=True)

        def cdrain(b):
            pltpu.make_async_copy(ones, cacc.at[didxs[b].at[0]],
                                  csem[b]).wait()

        def proc(ci, b, drain_prev=True, fire_next=True):
            bn = (b + 1) % 3
            if drain_prev:
                sdrain(bn)
                if with_counts:
                    cdrain(bn)
            if fire_next:
                ifire(ci + 1, bn)
                gfire(ci + 1, bn)
            gdrain(b)
            idrain(b)
            sfire(b)
            if with_counts:
                cfire(b)

        ifire(0, 0)
        gfire(0, 0)
        proc(0, 0, drain_prev=False)
        proc(1, 1, drain_prev=False)
        proc(2, 2)

        def triple(t, carry):
            c0 = 3 * t
            proc(c0, 0)
            proc(c0 + 1, 1)
            proc(c0 + 2, 2)
            return carry

        lax.fori_loop(1, (n_chunks - 8) // 3 + 1, triple, 0)

        proc(n_chunks - 5, 0)
        proc(n_chunks - 4, 1)
        proc(n_chunks - 3, 2)
        proc(n_chunks - 2, 0)
        proc(n_chunks - 1, 1, fire_next=False)
        for b in (0, 1):
            sdrain(b)
            if with_counts:
                cdrain(b)

        plsc.subcore_barrier()

        pltpu.sync_copy(acc.at[pl.ds(s * rpt, rpt)],
                        out_hbm.at[c, pl.ds(s * rpt, rpt)])

        if with_counts:
            @pl.when(jnp.logical_and(s == 0, c == 0))
            def _():
                pltpu.sync_copy(cacc, c0_hbm)

            @pl.when(jnp.logical_and(s == 0, c == 1))
            def _():
                pltpu.sync_copy(cacc, c1_hbm)

    res = agg(x, src, dstr3, z2, z1)
    if not with_counts and isinstance(res, (list, tuple)):
        return res[0]
    return res


def _tc_layer(parts, cnts, x, wlt, b2d, wrt, relu):
    NP, D = x.shape
    BN = 2048
    assert NP % BN == 0
    grid = NP // BN

    def body(part_ref, cnt_ref, x_ref, wl_ref, b_ref, wr_ref, o_ref):
        i = pl.program_id(0)
        csum = (cnt_ref[0, pl.ds(i * BN, BN)] +
                cnt_ref[1, pl.ds(i * BN, BN)])
        inv = 1.0 / jnp.maximum(csum, 1.0)
        agg = part_ref[0] + part_ref[1]
        mean = agg * inv[:, None]
        h = (jnp.dot(mean, wl_ref[...], preferred_element_type=jnp.float32)
             + b_ref[...]
             + jnp.dot(x_ref[...], wr_ref[...],
                       preferred_element_type=jnp.float32))
        if relu:
            h = jnp.maximum(h, 0.0)
        o_ref[...] = h

    return pl.pallas_call(
        body,
        grid=(grid,),
        in_specs=[
            pl.BlockSpec((2, BN, D), lambda i: (0, i, 0)),
            pl.BlockSpec((2, NP), lambda i: (0, 0)),
            pl.BlockSpec((BN, D), lambda i: (i, 0)),
            pl.BlockSpec((D, D), lambda i: (0, 0)),
            pl.BlockSpec((1, D), lambda i: (0, 0)),
            pl.BlockSpec((D, D), lambda i: (0, 0)),
        ],
        out_specs=pl.BlockSpec((BN, D), lambda i: (i, 0)),
        out_shape=jax.ShapeDtypeStruct((NP, D), jnp.float32),
    )(parts, cnts, x, wlt, b2d, wrt)


@jax.jit
def kernel(x, edge_index, W1l, b1, W1r, W2l, b2, W2r):
    N, D = x.shape
    NP = ((N + 2047) // 2048) * 2048
    NW = NC * NS
    E = edge_index.shape[1]
    n_chunks = E // (NW * K)
    src = edge_index[0]
    dstr3 = edge_index[1].reshape(NW * n_chunks, 1, K)
    xp = jnp.pad(x, ((0, NP - N), (0, 0)))
    z2 = jnp.zeros((NP, D), jnp.float32)
    z1 = jnp.zeros((NP,), jnp.float32)

    p1, c1a, c1b = _sc_agg(xp, src, dstr3, z2, z1, with_counts=True)
    cn1 = jnp.stack([c1a, c1b])
    h = _tc_layer(p1, cn1, xp, W1l.T, b1.reshape(1, D), W1r.T, relu=True)
    p2 = _sc_agg(h, src, dstr3, z2, z1, with_counts=False)
    out = _tc_layer(p2, cn1, h, W2l.T, b2.reshape(1, D), W2r.T, relu=False)
    return out[:N]


# speedup vs baseline: 14.5543x; 1.0915x over previous
import functools

import jax
import jax.numpy as jnp
from jax import lax
from jax.experimental import pallas as pl
from jax.experimental.pallas import tpu as pltpu
from jax.experimental.pallas import tpu_sc as plsc

NC = 2
NS = 16
K = 80


def _sc_agg(x, eflat, z2, z1, NP, with_counts):
    N, D = x.shape
    NW = NC * NS
    E = eflat.shape[0] // 2
    ept = E // NW
    n_chunks = ept // K
    rpt = NP // NS
    assert E % NW == 0 and ept % K == 0 and rpt % 8 == 0
    assert z2.shape == (rpt, D) and z1.shape == (NP,)
    assert n_chunks % 3 == 2 and n_chunks >= 8

    mesh = plsc.VectorSubcoreMesh(core_axis_name="c", subcore_axis_name="s")

    out_type = [jax.ShapeDtypeStruct((NC, NP, D), jnp.float32)]
    if with_counts:
        out_type += [jax.ShapeDtypeStruct((NP,), jnp.float32),
                     jax.ShapeDtypeStruct((NP,), jnp.float32)]

    @functools.partial(
        pl.kernel, mesh=mesh, out_type=out_type,
        scratch_types=[
            pltpu.VMEM((ept,), jnp.int32),
            pltpu.VMEM((K,), jnp.int32),
            pltpu.VMEM((K,), jnp.int32),
            pltpu.VMEM((K,), jnp.int32),
            pltpu.VMEM((K, D), jnp.float32),
            pltpu.VMEM((K, D), jnp.float32),
            pltpu.VMEM((K, D), jnp.float32),
            pltpu.VMEM((K,), jnp.float32),
            pltpu.VMEM_SHARED((NP, D), jnp.float32),
            pltpu.VMEM_SHARED((NP,), jnp.float32),
        ] + [pltpu.SemaphoreType.DMA] * 12,
    )
    def agg(*refs):
        if with_counts:
            (x_hbm, e_hbm, z2_hbm, z1_hbm,
             out_hbm, c0_hbm, c1_hbm, sidx,
             dx0, dx1, dx2, rw0, rw1, rw2, ones, acc, cacc, *sems) = refs
        else:
            (x_hbm, e_hbm, z2_hbm, z1_hbm, out_hbm, sidx,
             dx0, dx1, dx2, rw0, rw1, rw2, ones, acc, cacc, *sems) = refs
        didxs = [dx0, dx1, dx2]
        rows = [rw0, rw1, rw2]
        gsem = sems[0:3]
        ssem = sems[3:6]
        isem = sems[6:9]
        csem = sems[9:12]
        c = lax.axis_index("c")
        s = lax.axis_index("s")
        wid = c * NS + s
        ebase = wid * ept
        dbase = E + wid * ept

        zcp = pltpu.async_copy(z2_hbm, acc.at[pl.ds(s * rpt, rpt)], gsem[0])
        scp = pltpu.async_copy(e_hbm.at[pl.ds(ebase, ept)], sidx, gsem[1])
        if with_counts:
            @pl.when(s == 0)
            def _():
                pltpu.sync_copy(z1_hbm, cacc)

            for j in range(K // 16):
                ones[pl.ds(j * 16, 16)] = jnp.ones((16,), jnp.float32)
        zcp.wait()
        scp.wait()

        plsc.subcore_barrier()

        def ifire(ci, b):
            pltpu.async_copy(e_hbm.at[pl.ds(dbase + ci * K, K)],
                             didxs[b], isem[b])

        def idrain(b):
            pltpu.make_async_copy(e_hbm.at[pl.ds(0, K)],
                                  didxs[b], isem[b]).wait()

        def gfire(ci, b):
            pltpu.async_copy(x_hbm.at[sidx.at[pl.ds(ci * K, K)]],
                             rows[b], gsem[b])

        def gdrain(b):
            pltpu.make_async_copy(x_hbm.at[sidx.at[pl.ds(0, K)]],
                                  rows[b], gsem[b]).wait()

        def sfire(b):
            pltpu.async_copy(rows[b], acc.at[didxs[b]], ssem[b], add=True)

        def sdrain(b):
            pltpu.make_async_copy(rows[b], acc.at[didxs[b]], ssem[b]).wait()

        def cfire(b):
            pltpu.async_copy(ones, cacc.at[didxs[b]], csem[b], add=True)

        def cdrain(b):
            pltpu.make_async_copy(ones, cacc.at[didxs[b]], csem[b]).wait()

        def proc(ci, b, drain_prev=True, fire_next=True):
            bn = (b + 1) % 3
            if drain_prev:
                sdrain(bn)
                if with_counts:
                    cdrain(bn)
            if fire_next:
                ifire(ci + 1, bn)
                gfire(ci + 1, bn)
            gdrain(b)
            idrain(b)
            sfire(b)
            if with_counts:
                cfire(b)

        ifire(0, 0)
        gfire(0, 0)
        proc(0, 0, drain_prev=False)
        proc(1, 1, drain_prev=False)
        proc(2, 2)

        def triple(t, carry):
            c0 = 3 * t
            proc(c0, 0)
            proc(c0 + 1, 1)
            proc(c0 + 2, 2)
            return carry

        lax.fori_loop(1, (n_chunks - 8) // 3 + 1, triple, 0)

        proc(n_chunks - 5, 0)
        proc(n_chunks - 4, 1)
        proc(n_chunks - 3, 2)
        proc(n_chunks - 2, 0)
        proc(n_chunks - 1, 1, fire_next=False)
        for b in (0, 1):
            sdrain(b)
            if with_counts:
                cdrain(b)

        plsc.subcore_barrier()

        pltpu.sync_copy(acc.at[pl.ds(s * rpt, rpt)],
                        out_hbm.at[c, pl.ds(s * rpt, rpt)])

        if with_counts:
            @pl.when(jnp.logical_and(s == 0, c == 0))
            def _():
                pltpu.sync_copy(cacc, c0_hbm)

            @pl.when(jnp.logical_and(s == 0, c == 1))
            def _():
                pltpu.sync_copy(cacc, c1_hbm)

    res = agg(x, eflat, z2, z1)
    if not with_counts and isinstance(res, (list, tuple)):
        return res[0]
    return res


def _tc_layer(parts, cnt0, cnt1, x, wl, b2d, wr, relu):
    N, D = x.shape
    NP = parts.shape[1]
    BN = 2048
    grid = (N + BN - 1) // BN
    assert grid * BN <= NP

    def body(part_ref, c0_ref, c1_ref, x_ref, wl_ref, b_ref, wr_ref, o_ref):
        i = pl.program_id(0)
        csum = c0_ref[pl.ds(i * BN, BN)] + c1_ref[pl.ds(i * BN, BN)]
        inv = 1.0 / jnp.maximum(csum, 1.0)
        agg = part_ref[0] + part_ref[1]
        mean = agg * inv[:, None]
        dn = (((1,), (1,)), ((), ()))
        h = (lax.dot_general(mean, wl_ref[...], dn,
                             preferred_element_type=jnp.float32)
             + b_ref[...]
             + lax.dot_general(x_ref[...], wr_ref[...], dn,
                               preferred_element_type=jnp.float32))
        if relu:
            h = jnp.maximum(h, 0.0)
        o_ref[...] = h

    return pl.pallas_call(
        body,
        grid=(grid,),
        in_specs=[
            pl.BlockSpec((2, BN, D), lambda i: (0, i, 0)),
            pl.BlockSpec((NP,), lambda i: (0,)),
            pl.BlockSpec((NP,), lambda i: (0,)),
            pl.BlockSpec((BN, D), lambda i: (i, 0)),
            pl.BlockSpec((D, D), lambda i: (0, 0)),
            pl.BlockSpec((1, D), lambda i: (0, 0)),
            pl.BlockSpec((D, D), lambda i: (0, 0)),
        ],
        out_specs=pl.BlockSpec((BN, D), lambda i: (i, 0)),
        out_shape=jax.ShapeDtypeStruct((N, D), jnp.float32),
    )(parts, cnt0, cnt1, x, wl, b2d, wr)


@jax.jit
def kernel(x, edge_index, W1l, b1, W1r, W2l, b2, W2r):
    N, D = x.shape
    NP = ((N + 2047) // 2048) * 2048
    E = edge_index.shape[1]
    eflat = edge_index.reshape(2 * E)
    z2 = jnp.zeros((NP // NS, D), jnp.float32)
    z1 = jnp.zeros((NP,), jnp.float32)

    p1, c1a, c1b = _sc_agg(x, eflat, z2, z1, NP, with_counts=True)
    h = _tc_layer(p1, c1a, c1b, x, W1l, b1.reshape(1, D), W1r, relu=True)
    p2 = _sc_agg(h, eflat, z2, z1, NP, with_counts=False)
    out = _tc_layer(p2, c1a, c1b, h, W2l, b2.reshape(1, D), W2r, relu=False)
    return out
